# baseline (speedup 1.0000x reference)
import functools

import jax
import jax.numpy as jnp
from jax import lax
from jax.experimental import pallas as pl
from jax.experimental.pallas import tpu as pltpu
from jax.experimental.pallas import tpu_sc as plsc

N = 10000
E = 160000
H = 128
NUM_CONVS = 4

NC = 2
NS = 16
NW = NC * NS
CH = 64
NCH = 80
CCH = 80
CNCH = 64
CQN = 16
EW = NCH * CH
E_PAD = NW * EW
N_ACC = 10112
JUNK = N
RPT = N_ACC // NS
EB = 2048
NB = 1264


def _deg_body(deg_ref, table_ref, out_ref):
    deg = deg_ref[...]
    oh = (lax.broadcasted_iota(jnp.int32, (NB, H), 1) == deg).astype(jnp.float32)
    out_ref[...] = jnp.dot(oh, table_ref[...], preferred_element_type=jnp.float32)


def _deg_embed(deg2d, deg_table_pad):
    return pl.pallas_call(
        _deg_body,
        grid=(N_ACC // NB,),
        in_specs=[
            pl.BlockSpec((NB, 1), lambda i: (i, 0)),
            pl.BlockSpec((H, H), lambda i: (0, 0)),
        ],
        out_specs=pl.BlockSpec((NB, H), lambda i: (i, 0)),
        out_shape=jax.ShapeDtypeStruct((N_ACC, H), jnp.float32),
    )(deg2d, deg_table_pad)


def _ea_body(el_ref, et_ref, w1_ref, b1_ref, w2_ref, b2_ref, bond_ref, out_ref):
    el = el_ref[...]
    d = jnp.maximum(el * w1_ref[...] + b1_ref[...], 0.0)
    d = jnp.dot(d, w2_ref[...], preferred_element_type=jnp.float32) + b2_ref[...]
    et = et_ref[...]
    oh = (lax.broadcasted_iota(jnp.int32, (EB, H), 1) == et).astype(jnp.float32)
    out_ref[...] = d * jnp.dot(oh, bond_ref[...], preferred_element_type=jnp.float32)


def _edge_attr(el_pad, et_pad, ee_w1, ee_b1, ee_w2, ee_b2, bond_pad):
    const = pl.BlockSpec((H, H), lambda i: (0, 0))
    bias = pl.BlockSpec((1, H), lambda i: (0, 0))
    return pl.pallas_call(
        _ea_body,
        grid=(E_PAD // EB,),
        in_specs=[
            pl.BlockSpec((EB, 1), lambda i: (i, 0)),
            pl.BlockSpec((EB, 1), lambda i: (i, 0)),
            pl.BlockSpec((1, H), lambda i: (0, 0)),
            bias, const, bias, const,
        ],
        out_specs=pl.BlockSpec((EB, H), lambda i: (i, 0)),
        out_shape=jax.ShapeDtypeStruct((E_PAD, H), jnp.float32),
    )(el_pad, et_pad, ee_w1, ee_b1, ee_w2, ee_b2, bond_pad)


def _gin_body(x_ref, a0_ref, a1_ref, w1_ref, b1_ref, w2_ref, b2_ref, out_ref,
              *, relu_last):
    h = x_ref[...] + a0_ref[...] + a1_ref[...]
    h = jnp.maximum(
        jnp.dot(h, w1_ref[...], preferred_element_type=jnp.float32) + b1_ref[...], 0.0)
    h = jnp.dot(h, w2_ref[...], preferred_element_type=jnp.float32) + b2_ref[...]
    if relu_last:
        h = jnp.maximum(h, 0.0)
    out_ref[...] = h


def _gin_mlp(x, a0, a1, w1, b1, w2, b2, relu_last):
    blk = pl.BlockSpec((NB, H), lambda i: (i, 0))
    const = pl.BlockSpec((H, H), lambda i: (0, 0))
    bias = pl.BlockSpec((1, H), lambda i: (0, 0))
    return pl.pallas_call(
        functools.partial(_gin_body, relu_last=relu_last),
        grid=(N_ACC // NB,),
        in_specs=[blk, blk, blk, const, bias, const, bias],
        out_specs=blk,
        out_shape=jax.ShapeDtypeStruct((N_ACC, H), jnp.float32),
    )(x, a0, a1, w1, b1, w2, b2)


def _ginf_body(x_ref, a0_ref, a1_ref, w1_ref, b1_ref, w2_ref, b2_ref,
               w1a_ref, w1b_ref, ys_ref, yd_ref):
    h = x_ref[...] + a0_ref[...] + a1_ref[...]
    h = jnp.maximum(
        jnp.dot(h, w1_ref[...], preferred_element_type=jnp.float32) + b1_ref[...], 0.0)
    h = jnp.dot(h, w2_ref[...], preferred_element_type=jnp.float32) + b2_ref[...]
    ys_ref[...] = jnp.dot(h, w1a_ref[...], preferred_element_type=jnp.float32)
    yd_ref[...] = jnp.dot(h, w1b_ref[...], preferred_element_type=jnp.float32)


def _gin_mlp_final(x, a0, a1, w1, b1, w2, b2, w1a, w1b):
    blk = pl.BlockSpec((NB, H), lambda i: (i, 0))
    const = pl.BlockSpec((H, H), lambda i: (0, 0))
    bias = pl.BlockSpec((1, H), lambda i: (0, 0))
    return pl.pallas_call(
        _ginf_body,
        grid=(N_ACC // NB,),
        in_specs=[blk, blk, blk, const, bias, const, bias, const, const],
        out_specs=(blk, blk),
        out_shape=(jax.ShapeDtypeStruct((N_ACC, H), jnp.float32),
                   jax.ShapeDtypeStruct((N_ACC, H), jnp.float32)),
    )(x, a0, a1, w1, b1, w2, b2, w1a, w1b)


def _emlp_body(g_ref, ea_ref, el_ref, pdiff_ref,
               w1c_ref, b1_ref, w2_ref, b2_ref, w3_ref, b3_ref, out_ref):
    e1 = (g_ref[...]
          + jnp.dot(ea_ref[...], w1c_ref[...], preferred_element_type=jnp.float32)
          + b1_ref[...])
    e1 = jnp.maximum(e1, 0.0)
    e2 = jnp.maximum(
        jnp.dot(e1, w2_ref[...], preferred_element_type=jnp.float32) + b2_ref[...], 0.0)
    inv = jnp.sum(e2 * w3_ref[...], axis=1, keepdims=True) + b3_ref[0, 0]
    out_ref[...] = pdiff_ref[...] * (inv / el_ref[...])


def _edge_mlp(g, ea, el_pad, pdiff, w1c, b1, w2p, b2p, w3p, b3):
    eblk = pl.BlockSpec((EB, H), lambda i: (i, 0))
    col = pl.BlockSpec((EB, 1), lambda i: (i, 0))
    pblk = pl.BlockSpec((EB, 4), lambda i: (i, 0))
    const = pl.BlockSpec((H, H), lambda i: (0, 0))
    bias = pl.BlockSpec((1, H), lambda i: (0, 0))
    return pl.pallas_call(
        _emlp_body,
        grid=(E_PAD // EB,),
        in_specs=[eblk, eblk, col, pblk,
                  const, bias, const, bias, bias,
                  pl.BlockSpec((1, 1), lambda i: (0, 0))],
        out_specs=pblk,
        out_shape=jax.ShapeDtypeStruct((E_PAD, 4), jnp.float32),
    )(g, ea, el_pad, pdiff, w1c, b1, w2p, b2p, w3p, b3)


def _final_body(eq_ref, pos_ref, out_ref):
    out_ref[...] = jnp.sum(eq_ref[...], axis=0) + pos_ref[...]


def _final_add(eqparts3, posflat2):
    return pl.pallas_call(
        _final_body,
        out_shape=jax.ShapeDtypeStruct((4, N_ACC), jnp.float32),
    )(eqparts3, posflat2)


_MESH = plsc.VectorSubcoreMesh(core_axis_name="c", subcore_axis_name="s",
                               num_cores=NC, num_subcores=NS)


@functools.partial(
    pl.kernel,
    out_type=jax.ShapeDtypeStruct((NC, N_ACC, H), jnp.float32),
    mesh=_MESH,
    scratch_types=[
        pltpu.VMEM((CQN, CCH), jnp.int32),
        pltpu.VMEM((CQN, CCH), jnp.int32),
        pltpu.VMEM((2, CCH, H), jnp.float32),
        pltpu.VMEM((2, CCH, H), jnp.float32),
        pltpu.VMEM_SHARED((N_ACC, H), jnp.float32),
        pltpu.SemaphoreType.DMA((2,)),
        pltpu.SemaphoreType.DMA((2,)),
        pltpu.SemaphoreType.DMA((2,)),
    ],
)
def _sc_conv(x_h, ea_h, srcg_h, dsts_h, zz_h, out_h,
             idxg_v, idxs_v, rows_v, ea_v, acc_sh, gsem, esem, ssem):
    c = lax.axis_index("c")
    s = lax.axis_index("s")
    w = c * NS + s
    sl_out = pl.ds(s * RPT, RPT)
    pltpu.sync_copy(zz_h.at[sl_out], acc_sh.at[sl_out])
    plsc.subcore_barrier()

    for hf in range(CNCH // CQN):
        pltpu.sync_copy(srcg_h.at[w, pl.ds(hf * CQN, CQN)], idxg_v)
        pltpu.sync_copy(dsts_h.at[w, pl.ds(hf * CQN, CQN)], idxs_v)
        pltpu.async_copy(x_h.at[idxg_v.at[0]], rows_v.at[0], gsem.at[0])
        pltpu.async_copy(ea_h.at[w, hf * CQN], ea_v.at[0], esem.at[0])

        @pl.loop(0, CQN)
        def _chunk(j):
            b = lax.rem(j, 2)
            nb = 1 - b

            @pl.when(j < CQN - 1)
            def _issue_next():
                @pl.when(j >= 1)
                def _drain_scatter():
                    pltpu.make_async_copy(x_h.at[pl.ds(0, CCH)], rows_v.at[nb],
                                          ssem.at[nb]).wait()
                pltpu.async_copy(x_h.at[idxg_v.at[j + 1]], rows_v.at[nb],
                                 gsem.at[nb])
                pltpu.async_copy(ea_h.at[w, hf * CQN + j + 1], ea_v.at[nb],
                                 esem.at[nb])

            pltpu.make_async_copy(x_h.at[pl.ds(0, CCH)], rows_v.at[b],
                                  gsem.at[b]).wait()
            pltpu.make_async_copy(x_h.at[pl.ds(0, CCH)], ea_v.at[b],
                                  esem.at[b]).wait()

            @pl.loop(0, CCH)
            def _row(r):
                for u in range(H // 16):
                    sl = pl.ds(u * 16, 16)
                    rows_v[b, r, sl] = jnp.maximum(
                        rows_v[b, r, sl] + ea_v[b, r, sl], 0.0)

            pltpu.async_copy(rows_v.at[b], acc_sh.at[idxs_v.at[j]], ssem.at[b],
                             add=True)

        pltpu.make_async_copy(x_h.at[pl.ds(0, CCH)], rows_v.at[0],
                              ssem.at[0]).wait()
        pltpu.make_async_copy(x_h.at[pl.ds(0, CCH)], rows_v.at[1],
                              ssem.at[1]).wait()

    plsc.subcore_barrier()
    pltpu.sync_copy(acc_sh.at[sl_out], out_h.at[c, sl_out])


@functools.partial(
    pl.kernel,
    out_type=(
        jax.ShapeDtypeStruct((NW, NCH, CH, H), jnp.float32),
        jax.ShapeDtypeStruct((NW, NCH, CH * 4), jnp.float32),
    ),
    mesh=_MESH,
    scratch_types=[
        pltpu.VMEM((NCH, CH), jnp.int32),
        pltpu.VMEM((NCH, CH), jnp.int32),
        pltpu.VMEM((NCH * CH,), jnp.float32),
        pltpu.VMEM((NCH * CH,), jnp.float32),
        pltpu.VMEM((N_ACC * 4,), jnp.float32),
        pltpu.VMEM((6, CH, H), jnp.float32),
        pltpu.VMEM((2, CH * 4), jnp.float32),
        pltpu.SemaphoreType.DMA((6,)),
        pltpu.SemaphoreType.DMA((6,)),
        pltpu.SemaphoreType.DMA((2,)),
    ],
    compiler_params=pltpu.CompilerParams(needs_layout_passes=False),
)
def _sc_fin_gather(ys_h, yd_h, pp4_h, srcg_h, dstg_h, srcgf_h, dstgf_h,
                   g_h, pd_h,
                   idxs_v, idxd_v, idxsf_v, idxdf_v, ptab_v, rows_v, pdiff_v,
                   gsem, wsem, psem):
    c = lax.axis_index("c")
    s = lax.axis_index("s")
    w = c * NS + s
    pltpu.sync_copy(pp4_h, ptab_v)
    pltpu.sync_copy(srcg_h.at[w], idxs_v)
    pltpu.sync_copy(dstg_h.at[w], idxd_v)
    pltpu.sync_copy(srcgf_h.at[w], idxsf_v)
    pltpu.sync_copy(dstgf_h.at[w], idxdf_v)

    lane = lax.iota(jnp.int32, 16)
    lane4 = lax.rem(lane, 4)
    egrp = lax.div(lane, 4)

    pltpu.async_copy(ys_h.at[idxs_v.at[0]], rows_v.at[0], gsem.at[0])
    pltpu.async_copy(yd_h.at[idxd_v.at[0]], rows_v.at[1], gsem.at[1])
    pltpu.async_copy(ys_h.at[idxs_v.at[1]], rows_v.at[2], gsem.at[2])
    pltpu.async_copy(yd_h.at[idxd_v.at[1]], rows_v.at[3], gsem.at[3])

    @pl.loop(0, NCH)
    def _chunk(j):
        b = lax.rem(j, 2)
        bs = 2 * lax.rem(j, 3)

        @pl.when(j < NCH - 2)
        def _issue_next():
            nbs = 2 * lax.rem(j + 2, 3)

            @pl.when(j >= 1)
            def _drain_writes():
                pltpu.make_async_copy(rows_v.at[nbs], g_h.at[0, 0],
                                      wsem.at[nbs]).wait()
            pltpu.async_copy(ys_h.at[idxs_v.at[j + 2]], rows_v.at[nbs],
                             gsem.at[nbs])
            pltpu.async_copy(yd_h.at[idxd_v.at[j + 2]], rows_v.at[nbs + 1],
                             gsem.at[nbs + 1])

        pltpu.make_async_copy(ys_h.at[pl.ds(0, CH)], rows_v.at[bs],
                              gsem.at[bs]).wait()
        pltpu.make_async_copy(ys_h.at[pl.ds(0, CH)], rows_v.at[bs + 1],
                              gsem.at[bs + 1]).wait()

        @pl.loop(0, CH)
        def _row(r):
            for u in range(H // 16):
                sl = pl.ds(u * 16, 16)
                rows_v[bs, r, sl] = rows_v[bs, r, sl] + rows_v[bs + 1, r, sl]

        @pl.when(j >= 2)
        def _drain_pdiff():
            pltpu.make_async_copy(pdiff_v.at[b], pd_h.at[0, 0], psem.at[b]).wait()

        @pl.loop(0, CH // 4)
        def _grp(gidx):
            col = j * CH + 4 * gidx + egrp
            sn = plsc.bitcast(plsc.load_gather(idxsf_v, [col]), jnp.int32)
            dn = plsc.bitcast(plsc.load_gather(idxdf_v, [col]), jnp.int32)
            ps = plsc.load_gather(ptab_v, [sn * 4 + lane4])
            pd = plsc.load_gather(ptab_v, [dn * 4 + lane4])
            pdiff_v[b, pl.ds(gidx * 16, 16)] = ps - pd

        pltpu.async_copy(rows_v.at[bs], g_h.at[w, j], wsem.at[bs])
        pltpu.async_copy(pdiff_v.at[b], pd_h.at[w, j], psem.at[b])

    for sl in (2 * ((NCH - 3) % 3), 2 * ((NCH - 2) % 3), 2 * ((NCH - 1) % 3)):
        pltpu.make_async_copy(rows_v.at[sl], g_h.at[0, 0], wsem.at[sl]).wait()
    pltpu.make_async_copy(pdiff_v.at[0], pd_h.at[0, 0], psem.at[0]).wait()
    pltpu.make_async_copy(pdiff_v.at[1], pd_h.at[0, 0], psem.at[1]).wait()


@functools.partial(
    pl.kernel,
    out_type=jax.ShapeDtypeStruct((NW, N_ACC * 4), jnp.float32),
    mesh=_MESH,
    scratch_types=[
        pltpu.VMEM((NCH * CH,), jnp.float32),
        pltpu.VMEM((NCH * CH,), jnp.float32),
        pltpu.VMEM((2, CH * 4), jnp.float32),
        pltpu.VMEM((N_ACC * 4,), jnp.float32),
        pltpu.SemaphoreType.DMA((2,)),
    ],
    compiler_params=pltpu.CompilerParams(needs_layout_passes=False),
)
def _sc_eq_scatter(sc_h, srcs_h, dsts_h, out_h,
                   idxs_v, idxd_v, val_v, acc_v, vsem):
    c = lax.axis_index("c")
    s = lax.axis_index("s")
    w = c * NS + s
    pltpu.sync_copy(srcs_h.at[w], idxs_v)
    pltpu.sync_copy(dsts_h.at[w], idxd_v)

    lane = lax.iota(jnp.int32, 16)
    lane4 = lax.rem(lane, 4)
    egrp = lax.div(lane, 4)
    masks = [(egrp == t) for t in range(4)]
    zeros = jnp.zeros((16,), jnp.float32)

    @pl.loop(0, (N_ACC * 4) // 16)
    def _zero(i):
        acc_v[pl.ds(i * 16, 16)] = zeros

    pltpu.async_copy(sc_h.at[w, 0], val_v.at[0], vsem.at[0])

    @pl.loop(0, NCH)
    def _chunk(j):
        b = lax.rem(j, 2)

        @pl.when(j < NCH - 1)
        def _issue_next():
            pltpu.async_copy(sc_h.at[w, j + 1], val_v.at[1 - b], vsem.at[1 - b])

        pltpu.make_async_copy(sc_h.at[0, 0], val_v.at[b], vsem.at[b]).wait()

        @pl.loop(0, CH // 4)
        def _grp(gidx):
            col = j * CH + 4 * gidx + egrp
            sn = plsc.bitcast(plsc.load_gather(idxs_v, [col]), jnp.int32)
            dn = plsc.bitcast(plsc.load_gather(idxd_v, [col]), jnp.int32)
            sflat = sn * 4 + lane4
            dflat = dn * 4 + lane4
            v = val_v[b, pl.ds(gidx * 16, 16)]
            nv = -v
            for t in range(4):
                plsc.addupdate_scatter(acc_v, [sflat], v, mask=masks[t])
                plsc.addupdate_scatter(acc_v, [dflat], nv, mask=masks[t])

    pltpu.sync_copy(acc_v, out_h.at[w])


def kernel(node_emb, node_type, node_degree, pos, edge_index, edge_type,
           edge_length, batch, time_step, deg_table, bond_table,
           ee_w1, ee_b1, ee_w2, ee_b2, gin_w1, gin_b1, gin_w2, gin_b2,
           mlp_w1, mlp_b1, mlp_w2, mlp_b2, mlp_w3, mlp_b3):
    src = edge_index[0].astype(jnp.int32)
    dst = edge_index[1].astype(jnp.int32)
    pad = E_PAD - E
    zpad = jnp.zeros((pad,), jnp.int32)
    jpad = jnp.full((pad,), JUNK, jnp.int32)
    srcg = jnp.concatenate([src, zpad]).reshape(NW, NCH, CH)
    dstg = jnp.concatenate([dst, zpad]).reshape(NW, NCH, CH)
    srcs = jnp.concatenate([src, jpad]).reshape(NW, NCH, CH)
    dsts = jnp.concatenate([dst, jpad]).reshape(NW, NCH, CH)
    srcg_c = srcg.reshape(NW, CNCH, CCH)
    dsts_c = dsts.reshape(NW, CNCH, CCH)

    el_pad = jnp.concatenate(
        [edge_length.astype(jnp.float32),
         jnp.ones((pad, 1), jnp.float32)]).reshape(E_PAD, 1)
    et_pad = jnp.concatenate(
        [edge_type.astype(jnp.int32), zpad]).reshape(E_PAD, 1)

    deg2d = jnp.pad(node_degree.astype(jnp.int32), (0, N_ACC - N)).reshape(N_ACC, 1)
    posflat = jnp.pad(pos.astype(jnp.float32),
                      ((0, N_ACC - N), (0, 1))).reshape(N_ACC * 4)

    deg_pad = jnp.pad(deg_table, ((0, H - deg_table.shape[0]), (0, 0)))
    bond_pad = jnp.pad(bond_table, ((0, H - bond_table.shape[0]), (0, 0)))

    zeros_h = jnp.zeros((N_ACC, H), jnp.float32)

    ea = _edge_attr(el_pad, et_pad, ee_w1, ee_b1.reshape(1, H),
                    ee_w2, ee_b2.reshape(1, H), bond_pad)
    ea4 = ea.reshape(NW, CNCH, CCH, H)
    x = _deg_embed(deg2d, deg_pad)

    w1a, w1b, w1c = mlp_w1[:H], mlp_w1[H:2 * H], mlp_w1[2 * H:]
    for i in range(NUM_CONVS - 1):
        acc = _sc_conv(x, ea4, srcg_c, dsts_c, zeros_h)
        x = _gin_mlp(x, acc[0], acc[1],
                     gin_w1[i], gin_b1[i].reshape(1, H),
                     gin_w2[i], gin_b2[i].reshape(1, H),
                     relu_last=True)
    acc = _sc_conv(x, ea4, srcg_c, dsts_c, zeros_h)
    ys, yd = _gin_mlp_final(x, acc[0], acc[1],
                            gin_w1[3], gin_b1[3].reshape(1, H),
                            gin_w2[3], gin_b2[3].reshape(1, H), w1a, w1b)

    g4, pd4 = _sc_fin_gather(
        ys, yd, posflat, srcg, dstg,
        lax.bitcast_convert_type(srcg.reshape(NW, NCH * CH), jnp.float32),
        lax.bitcast_convert_type(dstg.reshape(NW, NCH * CH), jnp.float32))
    w2p = jnp.pad(mlp_w2, ((0, 0), (0, H - mlp_w2.shape[1])))
    b2p = jnp.pad(mlp_b2, (0, H - mlp_b2.shape[0])).reshape(1, H)
    w3v = jnp.pad(mlp_w3[:, 0], (0, H - mlp_w3.shape[0])).reshape(1, H)
    sc_vals = _edge_mlp(g4.reshape(E_PAD, H), ea, el_pad,
                        pd4.reshape(E_PAD, 4),
                        w1c, mlp_b1.reshape(1, H), w2p, b2p, w3v,
                        mlp_b3.reshape(1, 1))
    eqparts = _sc_eq_scatter(
        sc_vals.reshape(NW, NCH, CH * 4),
        lax.bitcast_convert_type(srcs.reshape(NW, NCH * CH), jnp.float32),
        lax.bitcast_convert_type(dsts.reshape(NW, NCH * CH), jnp.float32))
    outflat = _final_add(eqparts.reshape(NW, 4, N_ACC), posflat.reshape(4, N_ACC))
    return outflat.reshape(N_ACC, 4)[:N, :3]


# speedup vs baseline: 1.0172x; 1.0172x over previous
import functools

import jax
import jax.numpy as jnp
from jax import lax
from jax.experimental import pallas as pl
from jax.experimental.pallas import tpu as pltpu
from jax.experimental.pallas import tpu_sc as plsc

N = 10000
E = 160000
H = 128
NUM_CONVS = 4

NC = 2
NS = 16
NW = NC * NS
CH = 64
NCH = 80
CCH = 64
CNCH = 80
CQN = 40
EW = NCH * CH
E_PAD = NW * EW
N_ACC = 10112
JUNK = N
RPT = N_ACC // NS
EB = 2048
NB = 1264


def _deg_body(deg_ref, table_ref, out_ref):
    deg = deg_ref[...]
    oh = (lax.broadcasted_iota(jnp.int32, (NB, H), 1) == deg).astype(jnp.float32)
    out_ref[...] = jnp.dot(oh, table_ref[...], preferred_element_type=jnp.float32)


def _deg_embed(deg2d, deg_table_pad):
    return pl.pallas_call(
        _deg_body,
        grid=(N_ACC // NB,),
        in_specs=[
            pl.BlockSpec((NB, 1), lambda i: (i, 0)),
            pl.BlockSpec((H, H), lambda i: (0, 0)),
        ],
        out_specs=pl.BlockSpec((NB, H), lambda i: (i, 0)),
        out_shape=jax.ShapeDtypeStruct((N_ACC, H), jnp.float32),
    )(deg2d, deg_table_pad)


def _ea_body(el_ref, et_ref, w1_ref, b1_ref, w2_ref, b2_ref, bond_ref, out_ref):
    el = el_ref[...]
    d = jnp.maximum(el * w1_ref[...] + b1_ref[...], 0.0)
    d = jnp.dot(d, w2_ref[...], preferred_element_type=jnp.float32) + b2_ref[...]
    et = et_ref[...]
    oh = (lax.broadcasted_iota(jnp.int32, (EB, H), 1) == et).astype(jnp.float32)
    out_ref[...] = d * jnp.dot(oh, bond_ref[...], preferred_element_type=jnp.float32)


def _edge_attr(el_pad, et_pad, ee_w1, ee_b1, ee_w2, ee_b2, bond_pad):
    const = pl.BlockSpec((H, H), lambda i: (0, 0))
    bias = pl.BlockSpec((1, H), lambda i: (0, 0))
    return pl.pallas_call(
        _ea_body,
        grid=(E_PAD // EB,),
        in_specs=[
            pl.BlockSpec((EB, 1), lambda i: (i, 0)),
            pl.BlockSpec((EB, 1), lambda i: (i, 0)),
            pl.BlockSpec((1, H), lambda i: (0, 0)),
            bias, const, bias, const,
        ],
        out_specs=pl.BlockSpec((EB, H), lambda i: (i, 0)),
        out_shape=jax.ShapeDtypeStruct((E_PAD, H), jnp.float32),
    )(el_pad, et_pad, ee_w1, ee_b1, ee_w2, ee_b2, bond_pad)


def _gin_body(x_ref, a0_ref, a1_ref, w1_ref, b1_ref, w2_ref, b2_ref, out_ref,
              *, relu_last):
    h = x_ref[...] + a0_ref[...] + a1_ref[...]
    h = jnp.maximum(
        jnp.dot(h, w1_ref[...], preferred_element_type=jnp.float32) + b1_ref[...], 0.0)
    h = jnp.dot(h, w2_ref[...], preferred_element_type=jnp.float32) + b2_ref[...]
    if relu_last:
        h = jnp.maximum(h, 0.0)
    out_ref[...] = h


def _gin_mlp(x, a0, a1, w1, b1, w2, b2, relu_last):
    blk = pl.BlockSpec((NB, H), lambda i: (i, 0))
    const = pl.BlockSpec((H, H), lambda i: (0, 0))
    bias = pl.BlockSpec((1, H), lambda i: (0, 0))
    return pl.pallas_call(
        functools.partial(_gin_body, relu_last=relu_last),
        grid=(N_ACC // NB,),
        in_specs=[blk, blk, blk, const, bias, const, bias],
        out_specs=blk,
        out_shape=jax.ShapeDtypeStruct((N_ACC, H), jnp.float32),
    )(x, a0, a1, w1, b1, w2, b2)


def _ginf_body(x_ref, a0_ref, a1_ref, w1_ref, b1_ref, w2_ref, b2_ref,
               w1a_ref, w1b_ref, ys_ref, yd_ref):
    h = x_ref[...] + a0_ref[...] + a1_ref[...]
    h = jnp.maximum(
        jnp.dot(h, w1_ref[...], preferred_element_type=jnp.float32) + b1_ref[...], 0.0)
    h = jnp.dot(h, w2_ref[...], preferred_element_type=jnp.float32) + b2_ref[...]
    ys_ref[...] = jnp.dot(h, w1a_ref[...], preferred_element_type=jnp.float32)
    yd_ref[...] = jnp.dot(h, w1b_ref[...], preferred_element_type=jnp.float32)


def _gin_mlp_final(x, a0, a1, w1, b1, w2, b2, w1a, w1b):
    blk = pl.BlockSpec((NB, H), lambda i: (i, 0))
    const = pl.BlockSpec((H, H), lambda i: (0, 0))
    bias = pl.BlockSpec((1, H), lambda i: (0, 0))
    return pl.pallas_call(
        _ginf_body,
        grid=(N_ACC // NB,),
        in_specs=[blk, blk, blk, const, bias, const, bias, const, const],
        out_specs=(blk, blk),
        out_shape=(jax.ShapeDtypeStruct((N_ACC, H), jnp.float32),
                   jax.ShapeDtypeStruct((N_ACC, H), jnp.float32)),
    )(x, a0, a1, w1, b1, w2, b2, w1a, w1b)


def _emlp_body(g_ref, ea_ref, el_ref, pdiff_ref,
               w1c_ref, b1_ref, w2_ref, b2_ref, w3_ref, b3_ref, out_ref):
    e1 = (g_ref[...]
          + jnp.dot(ea_ref[...], w1c_ref[...], preferred_element_type=jnp.float32)
          + b1_ref[...])
    e1 = jnp.maximum(e1, 0.0)
    e2 = jnp.maximum(
        jnp.dot(e1, w2_ref[...], preferred_element_type=jnp.float32) + b2_ref[...], 0.0)
    inv = jnp.sum(e2 * w3_ref[...], axis=1, keepdims=True) + b3_ref[0, 0]
    out_ref[...] = pdiff_ref[...] * (inv / el_ref[...])


def _edge_mlp(g, ea, el_pad, pdiff, w1c, b1, w2p, b2p, w3p, b3):
    eblk = pl.BlockSpec((EB, H), lambda i: (i, 0))
    col = pl.BlockSpec((EB, 1), lambda i: (i, 0))
    pblk = pl.BlockSpec((EB, 4), lambda i: (i, 0))
    const = pl.BlockSpec((H, H), lambda i: (0, 0))
    bias = pl.BlockSpec((1, H), lambda i: (0, 0))
    return pl.pallas_call(
        _emlp_body,
        grid=(E_PAD // EB,),
        in_specs=[eblk, eblk, col, pblk,
                  const, bias, const, bias, bias,
                  pl.BlockSpec((1, 1), lambda i: (0, 0))],
        out_specs=pblk,
        out_shape=jax.ShapeDtypeStruct((E_PAD, 4), jnp.float32),
    )(g, ea, el_pad, pdiff, w1c, b1, w2p, b2p, w3p, b3)


def _final_body(eq_ref, pos_ref, out_ref):
    out_ref[...] = jnp.sum(eq_ref[...], axis=0) + pos_ref[...]


def _final_add(eqparts3, posflat2):
    return pl.pallas_call(
        _final_body,
        out_shape=jax.ShapeDtypeStruct((4, N_ACC), jnp.float32),
    )(eqparts3, posflat2)


_MESH = plsc.VectorSubcoreMesh(core_axis_name="c", subcore_axis_name="s",
                               num_cores=NC, num_subcores=NS)


@functools.partial(
    pl.kernel,
    out_type=jax.ShapeDtypeStruct((NC, N_ACC, H), jnp.float32),
    mesh=_MESH,
    scratch_types=[
        pltpu.VMEM((CQN, CCH), jnp.int32),
        pltpu.VMEM((CQN, CCH), jnp.int32),
        pltpu.VMEM((2, CCH, H), jnp.float32),
        pltpu.VMEM((2, CCH, H), jnp.float32),
        pltpu.VMEM_SHARED((N_ACC, H), jnp.float32),
        pltpu.SemaphoreType.DMA((2,)),
        pltpu.SemaphoreType.DMA((2,)),
        pltpu.SemaphoreType.DMA((2,)),
    ],
)
def _sc_conv(x_h, ea_h, srcg_h, dsts_h, zz_h, out_h,
             idxg_v, idxs_v, rows_v, ea_v, acc_sh, gsem, esem, ssem):
    c = lax.axis_index("c")
    s = lax.axis_index("s")
    w = c * NS + s
    sl_out = pl.ds(s * RPT, RPT)
    pltpu.sync_copy(zz_h.at[sl_out], acc_sh.at[sl_out])
    plsc.subcore_barrier()

    for hf in range(CNCH // CQN):
        pltpu.sync_copy(srcg_h.at[w, pl.ds(hf * CQN, CQN)], idxg_v)
        pltpu.sync_copy(dsts_h.at[w, pl.ds(hf * CQN, CQN)], idxs_v)
        pltpu.async_copy(x_h.at[idxg_v.at[0]], rows_v.at[0], gsem.at[0])
        pltpu.async_copy(ea_h.at[w, hf * CQN], ea_v.at[0], esem.at[0])

        @pl.loop(0, CQN)
        def _chunk(j):
            b = lax.rem(j, 2)
            nb = 1 - b

            @pl.when(j < CQN - 1)
            def _issue_next():
                @pl.when(j >= 1)
                def _drain_scatter():
                    pltpu.make_async_copy(x_h.at[pl.ds(0, CCH)], rows_v.at[nb],
                                          ssem.at[nb]).wait()
                pltpu.async_copy(x_h.at[idxg_v.at[j + 1]], rows_v.at[nb],
                                 gsem.at[nb])
                pltpu.async_copy(ea_h.at[w, hf * CQN + j + 1], ea_v.at[nb],
                                 esem.at[nb])

            pltpu.make_async_copy(x_h.at[pl.ds(0, CCH)], rows_v.at[b],
                                  gsem.at[b]).wait()
            pltpu.make_async_copy(x_h.at[pl.ds(0, CCH)], ea_v.at[b],
                                  esem.at[b]).wait()

            @pl.loop(0, CCH)
            def _row(r):
                for u in range(H // 16):
                    sl = pl.ds(u * 16, 16)
                    rows_v[b, r, sl] = jnp.maximum(
                        rows_v[b, r, sl] + ea_v[b, r, sl], 0.0)

            pltpu.async_copy(rows_v.at[b], acc_sh.at[idxs_v.at[j]], ssem.at[b],
                             add=True)

        pltpu.make_async_copy(x_h.at[pl.ds(0, CCH)], rows_v.at[0],
                              ssem.at[0]).wait()
        pltpu.make_async_copy(x_h.at[pl.ds(0, CCH)], rows_v.at[1],
                              ssem.at[1]).wait()

    plsc.subcore_barrier()
    pltpu.sync_copy(acc_sh.at[sl_out], out_h.at[c, sl_out])


@functools.partial(
    pl.kernel,
    out_type=(
        jax.ShapeDtypeStruct((NW, NCH, CH, H), jnp.float32),
        jax.ShapeDtypeStruct((NW, NCH, CH * 4), jnp.float32),
    ),
    mesh=_MESH,
    scratch_types=[
        pltpu.VMEM((NCH, CH), jnp.int32),
        pltpu.VMEM((NCH, CH), jnp.int32),
        pltpu.VMEM((NCH * CH,), jnp.float32),
        pltpu.VMEM((NCH * CH,), jnp.float32),
        pltpu.VMEM((N_ACC * 4,), jnp.float32),
        pltpu.VMEM((6, CH, H), jnp.float32),
        pltpu.VMEM((2, CH * 4), jnp.float32),
        pltpu.SemaphoreType.DMA((6,)),
        pltpu.SemaphoreType.DMA((6,)),
        pltpu.SemaphoreType.DMA((2,)),
    ],
    compiler_params=pltpu.CompilerParams(needs_layout_passes=False),
)
def _sc_fin_gather(ys_h, yd_h, pp4_h, srcg_h, dstg_h, srcgf_h, dstgf_h,
                   g_h, pd_h,
                   idxs_v, idxd_v, idxsf_v, idxdf_v, ptab_v, rows_v, pdiff_v,
                   gsem, wsem, psem):
    c = lax.axis_index("c")
    s = lax.axis_index("s")
    w = c * NS + s
    pltpu.sync_copy(pp4_h, ptab_v)
    pltpu.sync_copy(srcg_h.at[w], idxs_v)
    pltpu.sync_copy(dstg_h.at[w], idxd_v)
    pltpu.sync_copy(srcgf_h.at[w], idxsf_v)
    pltpu.sync_copy(dstgf_h.at[w], idxdf_v)

    lane = lax.iota(jnp.int32, 16)
    lane4 = lax.rem(lane, 4)
    egrp = lax.div(lane, 4)

    pltpu.async_copy(ys_h.at[idxs_v.at[0]], rows_v.at[0], gsem.at[0])
    pltpu.async_copy(yd_h.at[idxd_v.at[0]], rows_v.at[1], gsem.at[1])
    pltpu.async_copy(ys_h.at[idxs_v.at[1]], rows_v.at[2], gsem.at[2])
    pltpu.async_copy(yd_h.at[idxd_v.at[1]], rows_v.at[3], gsem.at[3])

    @pl.loop(0, NCH)
    def _chunk(j):
        b = lax.rem(j, 2)
        bs = 2 * lax.rem(j, 3)

        @pl.when(j < NCH - 2)
        def _issue_next():
            nbs = 2 * lax.rem(j + 2, 3)

            @pl.when(j >= 1)
            def _drain_writes():
                pltpu.make_async_copy(rows_v.at[nbs], g_h.at[0, 0],
                                      wsem.at[nbs]).wait()
            pltpu.async_copy(ys_h.at[idxs_v.at[j + 2]], rows_v.at[nbs],
                             gsem.at[nbs])
            pltpu.async_copy(yd_h.at[idxd_v.at[j + 2]], rows_v.at[nbs + 1],
                             gsem.at[nbs + 1])

        pltpu.make_async_copy(ys_h.at[pl.ds(0, CH)], rows_v.at[bs],
                              gsem.at[bs]).wait()
        pltpu.make_async_copy(ys_h.at[pl.ds(0, CH)], rows_v.at[bs + 1],
                              gsem.at[bs + 1]).wait()

        @pl.loop(0, CH)
        def _row(r):
            for u in range(H // 16):
                sl = pl.ds(u * 16, 16)
                rows_v[bs, r, sl] = rows_v[bs, r, sl] + rows_v[bs + 1, r, sl]

        @pl.when(j >= 2)
        def _drain_pdiff():
            pltpu.make_async_copy(pdiff_v.at[b], pd_h.at[0, 0], psem.at[b]).wait()

        @pl.loop(0, CH // 4)
        def _grp(gidx):
            col = j * CH + 4 * gidx + egrp
            sn = plsc.bitcast(plsc.load_gather(idxsf_v, [col]), jnp.int32)
            dn = plsc.bitcast(plsc.load_gather(idxdf_v, [col]), jnp.int32)
            ps = plsc.load_gather(ptab_v, [sn * 4 + lane4])
            pd = plsc.load_gather(ptab_v, [dn * 4 + lane4])
            pdiff_v[b, pl.ds(gidx * 16, 16)] = ps - pd

        pltpu.async_copy(rows_v.at[bs], g_h.at[w, j], wsem.at[bs])
        pltpu.async_copy(pdiff_v.at[b], pd_h.at[w, j], psem.at[b])

    for sl in (2 * ((NCH - 3) % 3), 2 * ((NCH - 2) % 3), 2 * ((NCH - 1) % 3)):
        pltpu.make_async_copy(rows_v.at[sl], g_h.at[0, 0], wsem.at[sl]).wait()
    pltpu.make_async_copy(pdiff_v.at[0], pd_h.at[0, 0], psem.at[0]).wait()
    pltpu.make_async_copy(pdiff_v.at[1], pd_h.at[0, 0], psem.at[1]).wait()


@functools.partial(
    pl.kernel,
    out_type=jax.ShapeDtypeStruct((NW, N_ACC * 4), jnp.float32),
    mesh=_MESH,
    scratch_types=[
        pltpu.VMEM((NCH * CH,), jnp.float32),
        pltpu.VMEM((NCH * CH,), jnp.float32),
        pltpu.VMEM((2, CH * 4), jnp.float32),
        pltpu.VMEM((N_ACC * 4,), jnp.float32),
        pltpu.SemaphoreType.DMA((2,)),
    ],
    compiler_params=pltpu.CompilerParams(needs_layout_passes=False),
)
def _sc_eq_scatter(sc_h, srcs_h, dsts_h, out_h,
                   idxs_v, idxd_v, val_v, acc_v, vsem):
    c = lax.axis_index("c")
    s = lax.axis_index("s")
    w = c * NS + s
    pltpu.sync_copy(srcs_h.at[w], idxs_v)
    pltpu.sync_copy(dsts_h.at[w], idxd_v)

    lane = lax.iota(jnp.int32, 16)
    lane4 = lax.rem(lane, 4)
    egrp = lax.div(lane, 4)
    masks = [(egrp == t) for t in range(4)]
    zeros = jnp.zeros((16,), jnp.float32)

    @pl.loop(0, (N_ACC * 4) // 16)
    def _zero(i):
        acc_v[pl.ds(i * 16, 16)] = zeros

    pltpu.async_copy(sc_h.at[w, 0], val_v.at[0], vsem.at[0])

    @pl.loop(0, NCH)
    def _chunk(j):
        b = lax.rem(j, 2)

        @pl.when(j < NCH - 1)
        def _issue_next():
            pltpu.async_copy(sc_h.at[w, j + 1], val_v.at[1 - b], vsem.at[1 - b])

        pltpu.make_async_copy(sc_h.at[0, 0], val_v.at[b], vsem.at[b]).wait()

        @pl.loop(0, CH // 4)
        def _grp(gidx):
            col = j * CH + 4 * gidx + egrp
            sn = plsc.bitcast(plsc.load_gather(idxs_v, [col]), jnp.int32)
            dn = plsc.bitcast(plsc.load_gather(idxd_v, [col]), jnp.int32)
            sflat = sn * 4 + lane4
            dflat = dn * 4 + lane4
            v = val_v[b, pl.ds(gidx * 16, 16)]
            nv = -v
            for t in range(4):
                plsc.addupdate_scatter(acc_v, [sflat], v, mask=masks[t])
                plsc.addupdate_scatter(acc_v, [dflat], nv, mask=masks[t])

    pltpu.sync_copy(acc_v, out_h.at[w])


def kernel(node_emb, node_type, node_degree, pos, edge_index, edge_type,
           edge_length, batch, time_step, deg_table, bond_table,
           ee_w1, ee_b1, ee_w2, ee_b2, gin_w1, gin_b1, gin_w2, gin_b2,
           mlp_w1, mlp_b1, mlp_w2, mlp_b2, mlp_w3, mlp_b3):
    src = edge_index[0].astype(jnp.int32)
    dst = edge_index[1].astype(jnp.int32)
    pad = E_PAD - E
    zpad = jnp.zeros((pad,), jnp.int32)
    jpad = jnp.full((pad,), JUNK, jnp.int32)
    srcg = jnp.concatenate([src, zpad]).reshape(NW, NCH, CH)
    dstg = jnp.concatenate([dst, zpad]).reshape(NW, NCH, CH)
    srcs = jnp.concatenate([src, jpad]).reshape(NW, NCH, CH)
    dsts = jnp.concatenate([dst, jpad]).reshape(NW, NCH, CH)
    srcg_c = srcg.reshape(NW, CNCH, CCH)
    dsts_c = dsts.reshape(NW, CNCH, CCH)

    el_pad = jnp.concatenate(
        [edge_length.astype(jnp.float32),
         jnp.ones((pad, 1), jnp.float32)]).reshape(E_PAD, 1)
    et_pad = jnp.concatenate(
        [edge_type.astype(jnp.int32), zpad]).reshape(E_PAD, 1)

    deg2d = jnp.pad(node_degree.astype(jnp.int32), (0, N_ACC - N)).reshape(N_ACC, 1)
    posflat = jnp.pad(pos.astype(jnp.float32),
                      ((0, N_ACC - N), (0, 1))).reshape(N_ACC * 4)

    deg_pad = jnp.pad(deg_table, ((0, H - deg_table.shape[0]), (0, 0)))
    bond_pad = jnp.pad(bond_table, ((0, H - bond_table.shape[0]), (0, 0)))

    zeros_h = jnp.zeros((N_ACC, H), jnp.float32)

    ea = _edge_attr(el_pad, et_pad, ee_w1, ee_b1.reshape(1, H),
                    ee_w2, ee_b2.reshape(1, H), bond_pad)
    ea4 = ea.reshape(NW, CNCH, CCH, H)
    x = _deg_embed(deg2d, deg_pad)

    w1a, w1b, w1c = mlp_w1[:H], mlp_w1[H:2 * H], mlp_w1[2 * H:]
    for i in range(NUM_CONVS - 1):
        acc = _sc_conv(x, ea4, srcg_c, dsts_c, zeros_h)
        x = _gin_mlp(x, acc[0], acc[1],
                     gin_w1[i], gin_b1[i].reshape(1, H),
                     gin_w2[i], gin_b2[i].reshape(1, H),
                     relu_last=True)
    acc = _sc_conv(x, ea4, srcg_c, dsts_c, zeros_h)
    ys, yd = _gin_mlp_final(x, acc[0], acc[1],
                            gin_w1[3], gin_b1[3].reshape(1, H),
                            gin_w2[3], gin_b2[3].reshape(1, H), w1a, w1b)

    g4, pd4 = _sc_fin_gather(
        ys, yd, posflat, srcg, dstg,
        lax.bitcast_convert_type(srcg.reshape(NW, NCH * CH), jnp.float32),
        lax.bitcast_convert_type(dstg.reshape(NW, NCH * CH), jnp.float32))
    w2p = jnp.pad(mlp_w2, ((0, 0), (0, H - mlp_w2.shape[1])))
    b2p = jnp.pad(mlp_b2, (0, H - mlp_b2.shape[0])).reshape(1, H)
    w3v = jnp.pad(mlp_w3[:, 0], (0, H - mlp_w3.shape[0])).reshape(1, H)
    sc_vals = _edge_mlp(g4.reshape(E_PAD, H), ea, el_pad,
                        pd4.reshape(E_PAD, 4),
                        w1c, mlp_b1.reshape(1, H), w2p, b2p, w3v,
                        mlp_b3.reshape(1, 1))
    eqparts = _sc_eq_scatter(
        sc_vals.reshape(NW, NCH, CH * 4),
        lax.bitcast_convert_type(srcs.reshape(NW, NCH * CH), jnp.float32),
        lax.bitcast_convert_type(dsts.reshape(NW, NCH * CH), jnp.float32))
    outflat = _final_add(eqparts.reshape(NW, 4, N_ACC), posflat.reshape(4, N_ACC))
    return outflat.reshape(N_ACC, 4)[:N, :3]
